# baseline (speedup 1.0000x reference)
import functools

import jax
import jax.numpy as jnp
from jax.experimental import pallas as pl

_T = 512
_C = 96


def _tome_body(x_ref, out_ref):
    x = x_ref[0]
    pairs = x[1:1025].reshape(_T, 2, _C)
    xb = pairs[:, 0, :]
    xa = pairs[:, 1, :]
    na = xa / jnp.sqrt(jnp.sum(xa * xa, axis=-1, keepdims=True))
    nb = xb / jnp.sqrt(jnp.sum(xb * xb, axis=-1, keepdims=True))
    scores = jax.lax.dot_general(
        na, nb, (((1,), (1,)), ((), ())), preferred_element_type=jnp.float32
    )
    mx = jnp.max(scores, axis=-1, keepdims=True)
    col = jax.lax.broadcasted_iota(jnp.int32, (_T, _T), 1)
    d = jnp.min(jnp.where(scores == mx, col, _T), axis=-1)
    onehot = (col == d[:, None]).astype(jnp.float32)
    acc = jax.lax.dot_general(
        onehot, xa, (((0,), (0,)), ((), ())), preferred_element_type=jnp.float32
    )
    cnt = jnp.sum(onehot, axis=0)
    out_ref[0, 0:1, :] = x[0:1, :]
    out_ref[0, 1:, :] = (xb + acc) / (1.0 + cnt)[:, None]


@functools.partial(jax.jit, static_argnames=("interpret",))
def kernel(hidden_states, interpret=False):
    B, T, C = hidden_states.shape
    t = (T - 1) // 2
    return pl.pallas_call(
        _tome_body,
        grid=(B,),
        in_specs=[pl.BlockSpec((1, T, C), lambda i: (i, 0, 0))],
        out_specs=pl.BlockSpec((1, t + 1, C), lambda i: (i, 0, 0)),
        out_shape=jax.ShapeDtypeStruct((B, t + 1, C), hidden_states.dtype),
        interpret=interpret,
    )(hidden_states)


# speedup vs baseline: 1.1168x; 1.1168x over previous
import functools

import jax
import jax.numpy as jnp
from jax.experimental import pallas as pl

_T = 512
_C = 96


def _tome_body(cls_ref, x_ref, out_ref):
    x = x_ref[0]
    xb = x[:, 0, :]
    xa = x[:, 1, :]
    na = xa / jnp.sqrt(jnp.sum(xa * xa, axis=-1, keepdims=True))
    nb = xb / jnp.sqrt(jnp.sum(xb * xb, axis=-1, keepdims=True))
    scores = jax.lax.dot_general(
        na, nb, (((1,), (1,)), ((), ())), preferred_element_type=jnp.float32
    )
    mx = jnp.max(scores, axis=-1, keepdims=True)
    col = jax.lax.broadcasted_iota(jnp.int32, (_T, _T), 1)
    d = jnp.min(jnp.where(scores == mx, col, _T), axis=-1)
    onehot = (col == d[:, None]).astype(jnp.float32)
    acc = jax.lax.dot_general(
        onehot, xa, (((0,), (0,)), ((), ())), preferred_element_type=jnp.float32
    )
    cnt = jnp.sum(onehot, axis=0)
    out_ref[0, 0:1, :] = cls_ref[0]
    out_ref[0, 1:, :] = (xb + acc) / (1.0 + cnt)[:, None]


@functools.partial(jax.jit, static_argnames=("interpret",))
def kernel(hidden_states, interpret=False):
    B, T, C = hidden_states.shape
    t = (T - 1) // 2
    cls = hidden_states[:, :1]
    pairs = hidden_states[:, 1:].reshape(B, t, 2, C)
    return pl.pallas_call(
        _tome_body,
        grid=(B,),
        in_specs=[
            pl.BlockSpec((1, 1, C), lambda i: (i, 0, 0)),
            pl.BlockSpec((1, t, 2, C), lambda i: (i, 0, 0, 0)),
        ],
        out_specs=pl.BlockSpec((1, t + 1, C), lambda i: (i, 0, 0)),
        out_shape=jax.ShapeDtypeStruct((B, t + 1, C), hidden_states.dtype),
        interpret=interpret,
    )(cls, pairs)


# speedup vs baseline: 1.5076x; 1.3500x over previous
import functools

import jax
import jax.numpy as jnp
from jax.experimental import pallas as pl

_T = 512
_C = 96


def _tome_body(cls_ref, xa_ref, xb_ref, out_ref):
    xa = xa_ref[0]
    xb = xb_ref[0]
    na = xa / jnp.sqrt(jnp.sum(xa * xa, axis=-1, keepdims=True))
    nb = xb / jnp.sqrt(jnp.sum(xb * xb, axis=-1, keepdims=True))
    scores = jax.lax.dot_general(
        na, nb, (((1,), (1,)), ((), ())), preferred_element_type=jnp.float32
    )
    mx = jnp.max(scores, axis=-1, keepdims=True)
    col = jax.lax.broadcasted_iota(jnp.int32, (_T, _T), 1)
    d = jnp.min(jnp.where(scores == mx, col, _T), axis=-1)
    onehot = (col == d[:, None]).astype(jnp.float32)
    acc = jax.lax.dot_general(
        onehot, xa, (((0,), (0,)), ((), ())), preferred_element_type=jnp.float32
    )
    cnt = jnp.sum(onehot, axis=0)
    out_ref[0, 0:1, :] = cls_ref[0]
    out_ref[0, 1:, :] = (xb + acc) / (1.0 + cnt)[:, None]


@functools.partial(jax.jit, static_argnames=("interpret",))
def kernel(hidden_states, interpret=False):
    B, T, C = hidden_states.shape
    t = (T - 1) // 2
    cls = hidden_states[:, :1]
    pairs = hidden_states[:, 1:].reshape(B, t, 2, C)
    xb = pairs[:, :, 0]
    xa = pairs[:, :, 1]
    return pl.pallas_call(
        _tome_body,
        grid=(B,),
        in_specs=[
            pl.BlockSpec((1, 1, C), lambda i: (i, 0, 0)),
            pl.BlockSpec((1, t, C), lambda i: (i, 0, 0)),
            pl.BlockSpec((1, t, C), lambda i: (i, 0, 0)),
        ],
        out_specs=pl.BlockSpec((1, t + 1, C), lambda i: (i, 0, 0)),
        out_shape=jax.ShapeDtypeStruct((B, t + 1, C), hidden_states.dtype),
        interpret=interpret,
    )(cls, xa, xb)
